# baseline (speedup 1.0000x reference)
import functools

import jax
import jax.numpy as jnp
from jax import lax
from jax.experimental import pallas as pl
from jax.experimental.pallas import tpu as pltpu
from jax.experimental.pallas import tpu_sc as plsc

NUM_EXPERTS = 16
TOP_K = 2
_NEG_INF = float("-inf")
TB = 1024
NW = 32
LANES = 16


def _gate_block(x_ref, w_ref, logits_ref):
    logits_ref[...] = lax.dot_general(
        w_ref[...], x_ref[...], (((1,), (1,)), ((), ())),
        preferred_element_type=jnp.float32)


def _sc_router(nt, logits_hbm, probs_hbm, tk_hbm, mask_hbm,
               lbuf, pbuf, tkbuf, mbuf):
    per_w = nt // NW
    w = lax.axis_index("s") * 2 + lax.axis_index("c")
    base = w * per_w
    pltpu.sync_copy(logits_hbm.at[:, pl.ds(base, per_w)], lbuf)
    zero = jnp.zeros((LANES,), jnp.float32)

    def body(c, carry):
        off = c * LANES
        vs = [lbuf[e, pl.ds(off, LANES)] for e in range(NUM_EXPERTS)]
        m1 = vs[0]
        i1 = jnp.zeros((LANES,), jnp.int32)
        for e in range(1, NUM_EXPERTS):
            gt = vs[e] > m1
            m1 = jnp.where(gt, vs[e], m1)
            i1 = jnp.where(gt, e, i1)
        m2 = jnp.full((LANES,), _NEG_INF, jnp.float32)
        i2 = jnp.zeros((LANES,), jnp.int32)
        for e in range(NUM_EXPERTS):
            cand = jnp.where(i1 == e, _NEG_INF, vs[e])
            gt = cand > m2
            m2 = jnp.where(gt, cand, m2)
            i2 = jnp.where(gt, e, i2)
        r = jnp.exp(m2 - m1)
        denom = 1.0 + r
        a = 1.0 / denom
        b = r / denom
        for e in range(NUM_EXPERTS):
            k1 = i1 == e
            k2 = i2 == e
            pbuf[e, pl.ds(off, LANES)] = jnp.where(k1, a, jnp.where(k2, b, zero))
            mbuf[0, e, pl.ds(off, LANES)] = jnp.where(k1, 1.0, 0.0)
            mbuf[1, e, pl.ds(off, LANES)] = jnp.where(k2, 1.0, 0.0)
        tkbuf[0, pl.ds(off, LANES)] = m1
        tkbuf[1, pl.ds(off, LANES)] = m2
        tkbuf[2, pl.ds(off, LANES)] = plsc.bitcast(i1, jnp.float32)
        tkbuf[3, pl.ds(off, LANES)] = plsc.bitcast(i2, jnp.float32)
        for row in range(4, 8):
            tkbuf[row, pl.ds(off, LANES)] = zero
        return carry

    lax.fori_loop(0, per_w // LANES, body, 0)
    pltpu.sync_copy(pbuf, probs_hbm.at[:, pl.ds(base, per_w)])
    pltpu.sync_copy(tkbuf, tk_hbm.at[:, pl.ds(base, per_w)])
    pltpu.sync_copy(mbuf, mask_hbm.at[:, :, pl.ds(base, per_w)])


def kernel(x, W_gate):
    Bsz, Tlen, D = x.shape
    E = W_gate.shape[0]
    nt = Bsz * Tlen
    xf = x.reshape(nt, D)
    grid = (nt // TB,)
    logitsT = pl.pallas_call(
        _gate_block,
        grid=grid,
        in_specs=[
            pl.BlockSpec((TB, D), lambda i: (i, 0)),
            pl.BlockSpec((E, D), lambda i: (0, 0)),
        ],
        out_specs=pl.BlockSpec((E, TB), lambda i: (0, i)),
        out_shape=jax.ShapeDtypeStruct((E, nt), jnp.float32),
        compiler_params=pltpu.CompilerParams(
            dimension_semantics=("parallel",)),
    )(xf, W_gate)

    per_w = nt // NW
    sc_fn = functools.partial(
        pl.kernel,
        out_type=[
            jax.ShapeDtypeStruct((E, nt), jnp.float32),
            jax.ShapeDtypeStruct((8, nt), jnp.float32),
            jax.ShapeDtypeStruct((TOP_K, E, nt), jnp.float32),
        ],
        mesh=plsc.VectorSubcoreMesh(core_axis_name="c", subcore_axis_name="s"),
        compiler_params=pltpu.CompilerParams(needs_layout_passes=False),
        scratch_types=[
            pltpu.VMEM((E, per_w), jnp.float32),
            pltpu.VMEM((E, per_w), jnp.float32),
            pltpu.VMEM((8, per_w), jnp.float32),
            pltpu.VMEM((TOP_K, E, per_w), jnp.float32),
        ],
    )(functools.partial(_sc_router, nt))
    probsT, tk, maskT = sc_fn(logitsT)

    probs = probsT.T.reshape(Bsz, Tlen, E)
    tkl = tk[0:TOP_K].T.reshape(Bsz, Tlen, TOP_K)
    tki = lax.bitcast_convert_type(
        tk[TOP_K:2 * TOP_K], jnp.int32).T.reshape(Bsz, Tlen, TOP_K)
    mask = maskT.transpose(0, 2, 1)
    return probs, tkl, tki, mask


# speedup vs baseline: 1.4150x; 1.4150x over previous
import jax
import jax.numpy as jnp
from jax import lax
from jax.experimental import pallas as pl
from jax.experimental.pallas import tpu as pltpu

NUM_EXPERTS = 16
TOP_K = 2
_NEG_INF = float("-inf")
TB = 1024


def _router_block(x_ref, w_ref, probs_ref, tk_ref, mask_ref):
    xb = x_ref[...]
    w = w_ref[...]
    logits = lax.dot_general(w, xb, (((1,), (1,)), ((), ())),
                             preferred_element_type=jnp.float32)
    iota = lax.broadcasted_iota(jnp.int32, logits.shape, 0)
    m1 = jnp.max(logits, axis=0, keepdims=True)
    i1 = jnp.min(jnp.where(logits == m1, iota, NUM_EXPERTS),
                 axis=0, keepdims=True)
    sel1 = iota == i1
    masked = jnp.where(sel1, _NEG_INF, logits)
    m2 = jnp.max(masked, axis=0, keepdims=True)
    i2 = jnp.min(jnp.where(masked == m2, iota, NUM_EXPERTS),
                 axis=0, keepdims=True)
    sel2 = iota == i2
    keep = sel1 | sel2
    e = jnp.exp(logits - m1)
    denom = 1.0 + jnp.exp(m2 - m1)
    probs_ref[...] = jnp.where(keep, e / denom, 0.0)
    tk_ref[...] = jnp.concatenate(
        [m1, m2,
         lax.bitcast_convert_type(i1, jnp.float32),
         lax.bitcast_convert_type(i2, jnp.float32),
         jnp.zeros((4, logits.shape[1]), jnp.float32)], axis=0)
    mask_ref[0] = sel1.astype(jnp.float32)
    mask_ref[1] = sel2.astype(jnp.float32)


def kernel(x, W_gate):
    Bsz, Tlen, D = x.shape
    E = W_gate.shape[0]
    nt = Bsz * Tlen
    xf = x.reshape(nt, D)
    grid = (nt // TB,)
    probsT, tk, maskT = pl.pallas_call(
        _router_block,
        grid=grid,
        in_specs=[
            pl.BlockSpec((TB, D), lambda i: (i, 0)),
            pl.BlockSpec((E, D), lambda i: (0, 0)),
        ],
        out_specs=[
            pl.BlockSpec((E, TB), lambda i: (0, i)),
            pl.BlockSpec((8, TB), lambda i: (0, i)),
            pl.BlockSpec((TOP_K, E, TB), lambda i: (0, 0, i)),
        ],
        out_shape=[
            jax.ShapeDtypeStruct((E, nt), jnp.float32),
            jax.ShapeDtypeStruct((8, nt), jnp.float32),
            jax.ShapeDtypeStruct((TOP_K, E, nt), jnp.float32),
        ],
        compiler_params=pltpu.CompilerParams(
            dimension_semantics=("parallel",)),
    )(xf, W_gate)
    probs = probsT.T.reshape(Bsz, Tlen, E)
    tkl = tk[0:TOP_K].T.reshape(Bsz, Tlen, TOP_K)
    tki = lax.bitcast_convert_type(
        tk[TOP_K:2 * TOP_K], jnp.int32).T.reshape(Bsz, Tlen, TOP_K)
    mask = maskT.transpose(0, 2, 1)
    return probs, tkl, tki, mask


# speedup vs baseline: 1.4402x; 1.0178x over previous
import jax
import jax.numpy as jnp
from jax import lax
from jax.experimental import pallas as pl
from jax.experimental.pallas import tpu as pltpu

NUM_EXPERTS = 16
TOP_K = 2
_NEG_INF = float("-inf")
TB = 1024


def _router_block(x_ref, w_ref, probs_ref, tk_ref, mask_ref):
    xb = x_ref[...]
    w = w_ref[...]
    logits = lax.dot_general(w, xb, (((1,), (1,)), ((), ())),
                             preferred_element_type=jnp.float32)
    iota = lax.broadcasted_iota(jnp.int32, logits.shape, 0)
    m1 = jnp.max(logits, axis=0, keepdims=True)
    i1 = jnp.min(jnp.where(logits == m1, iota, NUM_EXPERTS),
                 axis=0, keepdims=True)
    sel1 = iota == i1
    masked = jnp.where(sel1, _NEG_INF, logits)
    m2 = jnp.max(masked, axis=0, keepdims=True)
    i2 = jnp.min(jnp.where(masked == m2, iota, NUM_EXPERTS),
                 axis=0, keepdims=True)
    sel2 = iota == i2
    keep = sel1 | sel2
    e = jnp.exp(logits - m1)
    denom = 1.0 + jnp.exp(m2 - m1)
    probs_ref[...] = jnp.where(keep, e / denom, 0.0)
    tk_ref[...] = jnp.concatenate(
        [m1, m2,
         lax.bitcast_convert_type(i1, jnp.float32),
         lax.bitcast_convert_type(i2, jnp.float32),
         jnp.zeros((4, logits.shape[1]), jnp.float32)], axis=0)
    mask_ref[0] = sel1.astype(jnp.float32)
    mask_ref[1] = sel2.astype(jnp.float32)


def kernel(x, W_gate):
    Bsz, Tlen, D = x.shape
    E = W_gate.shape[0]
    nt = Bsz * Tlen
    xf = x.reshape(nt, D)
    grid = (nt // TB,)
    probsT, tk, maskT = pl.pallas_call(
        _router_block,
        grid=grid,
        in_specs=[
            pl.BlockSpec((TB, D), lambda i: (i, 0)),
            pl.BlockSpec((E, D), lambda i: (0, 0)),
        ],
        out_specs=[
            pl.BlockSpec((E, TB), lambda i: (0, i)),
            pl.BlockSpec((8, TB), lambda i: (0, i)),
            pl.BlockSpec((TOP_K, E, TB), lambda i: (0, 0, i)),
        ],
        out_shape=[
            jax.ShapeDtypeStruct((E, nt), jnp.float32),
            jax.ShapeDtypeStruct((8, nt), jnp.float32),
            jax.ShapeDtypeStruct((TOP_K, E, nt), jnp.float32),
        ],
        compiler_params=pltpu.CompilerParams(
            dimension_semantics=("arbitrary",)),
    )(xf, W_gate)
    probs = probsT.T.reshape(Bsz, Tlen, E)
    tkl = tk[0:TOP_K].T.reshape(Bsz, Tlen, TOP_K)
    tki = lax.bitcast_convert_type(
        tk[TOP_K:2 * TOP_K], jnp.int32).T.reshape(Bsz, Tlen, TOP_K)
    mask = maskT.transpose(0, 2, 1)
    return probs, tkl, tki, mask
